# Initial kernel scaffold; baseline (speedup 1.0000x reference)
import jax
import jax.numpy as jnp
from jax.experimental import pallas as pl


def kernel(x, emb, fm_w, fm_b, W1, b1, W2, b2, W3, b3, Wout):
    raise NotImplementedError("write your pallas kernel here")


# speedup vs baseline: 30.0924x; 30.0924x over previous
import functools

import jax
import jax.numpy as jnp
from jax import lax
from jax.experimental import pallas as pl
from jax.experimental.pallas import tpu as pltpu
from jax.experimental.pallas import tpu_sc as plsc

B = 4096
F = 26
V = 1000
D = 16
H = 128

NC = 2
NS = 16
NW = NC * NS
RW = B // NW
GW = RW * F
NCHUNK = GW // 128
RCHUNK = RW // 16


def _sc_body(emb_hbm, fidx_hbm, x_hbm, w_hbm,
             dense_hbm, wsum_hbm,
             idx_v, xv, rows_v, wv, wsum_v, sem):
    c = lax.axis_index("c")
    s = lax.axis_index("s")
    wid = s * NC + c

    pltpu.sync_copy(fidx_hbm.at[wid], idx_v)
    pltpu.sync_copy(x_hbm.at[wid], xv)
    pltpu.sync_copy(w_hbm, wv)

    copies = []
    for j in range(NCHUNK):
        copies.append(
            pltpu.async_copy(emb_hbm.at[idx_v.at[j]],
                             rows_v.at[pl.ds(j * 128, 128)], sem))

    def chunk_body(r, _):
        base = pl.multiple_of(r * (16 * F), 16)
        lane = lax.iota(jnp.int32, 16) * F
        vals = [plsc.load_gather(xv, [lane + (base + f)]) for f in range(F)]
        wsum = jnp.zeros((16,), jnp.float32)
        for f in range(F):
            wf = plsc.load_gather(wv, [vals[f]])
            if f == 0:
                wsum = wf
            else:
                dup = vals[f] == vals[0]
                for f2 in range(1, f):
                    dup = dup | (vals[f] == vals[f2])
                wsum = wsum + jnp.where(dup, 0.0, wf)
        wsum_v[pl.ds(pl.multiple_of(r * 16, 16), 16)] = wsum
        return _

    lax.fori_loop(0, RCHUNK, chunk_body, None)

    for cp in copies:
        cp.wait()
    pltpu.sync_copy(rows_v, dense_hbm.at[pl.ds(pl.multiple_of(wid * GW, 8), GW)])
    pltpu.sync_copy(wsum_v, wsum_hbm.at[pl.ds(pl.multiple_of(wid * RW, 8), RW)])


@jax.jit
def _sc_gather_linear(emb_flat, fidx, xflat, w1k):
    mesh = plsc.VectorSubcoreMesh(core_axis_name="c", subcore_axis_name="s")
    run = pl.kernel(
        _sc_body,
        out_type=(
            jax.ShapeDtypeStruct((B * F, D), jnp.float32),
            jax.ShapeDtypeStruct((B,), jnp.float32),
        ),
        mesh=mesh,
        compiler_params=pltpu.CompilerParams(
            needs_layout_passes=False, use_tc_tiling_on_sc=False),
        scratch_types=[
            pltpu.VMEM((NCHUNK, 128), jnp.int32),
            pltpu.VMEM((GW,), jnp.int32),
            pltpu.VMEM((GW, D), jnp.float32),
            pltpu.VMEM((V,), jnp.float32),
            pltpu.VMEM((RW,), jnp.float32),
            pltpu.SemaphoreType.DMA,
        ],
    )
    return run(emb_flat, fidx, xflat, w1k)


def _tc_body(dense_ref, wsum_ref, fmb_ref, w1_ref, b1_ref, w2_ref, b2_ref,
             w3_ref, b3_ref, wout_ref, out_ref):
    d = dense_ref[:]
    ssum = d[:, 0:D]
    sq = d[:, 0:D] * d[:, 0:D]
    for f in range(1, F):
        blk = d[:, f * D:(f + 1) * D]
        ssum = ssum + blk
        sq = sq + blk * blk
    pair = 0.5 * jnp.sum(ssum * ssum - sq, axis=1, keepdims=True)

    h = jnp.maximum(
        jnp.dot(d, w1_ref[:], preferred_element_type=jnp.float32) + b1_ref[:], 0.0)
    h = jnp.maximum(
        jnp.dot(h, w2_ref[:], preferred_element_type=jnp.float32) + b2_ref[:], 0.0)
    h = jnp.maximum(
        jnp.dot(h, w3_ref[:], preferred_element_type=jnp.float32) + b3_ref[:], 0.0)
    y = jnp.dot(h, wout_ref[:], preferred_element_type=jnp.float32)
    out_ref[:] = y + pair + wsum_ref[:] + fmb_ref[0, 0]


TC_BLK = 512


@jax.jit
def _tc_fm_mlp(dense2d, wsum2d, fmb2d, W1t, b1r, W2t, b2r, W3t, b3r, Woutt):
    grid = (B // TC_BLK,)
    full = lambda shape: pl.BlockSpec(shape, lambda i: (0, 0))
    return pl.pallas_call(
        _tc_body,
        grid=grid,
        in_specs=[
            pl.BlockSpec((TC_BLK, F * D), lambda i: (i, 0)),
            pl.BlockSpec((TC_BLK, 1), lambda i: (i, 0)),
            full((1, 1)),
            full((F * D, H)),
            full((1, H)),
            full((H, H)),
            full((1, H)),
            full((H, H)),
            full((1, H)),
            full((H, 1)),
        ],
        out_specs=pl.BlockSpec((TC_BLK, 1), lambda i: (i, 0)),
        out_shape=jax.ShapeDtypeStruct((B, 1), jnp.float32),
    )(dense2d, wsum2d, fmb2d, W1t, b1r, W2t, b2r, W3t, b3r, Woutt)


def kernel(x, emb, fm_w, fm_b, W1, b1, W2, b2, W3, b3, Wout):
    offsets = jnp.arange(F, dtype=jnp.int32) * V
    fidx = (x + offsets[None, :]).reshape(NW, NCHUNK, 128)
    xflat = x.reshape(NW, GW)
    emb_flat = emb.reshape(F * V, D)
    w1k = fm_w[0, :V]

    dense, wsum = _sc_gather_linear(emb_flat, fidx, xflat, w1k)

    y = _tc_fm_mlp(
        dense.reshape(B, F * D),
        wsum.reshape(B, 1),
        fm_b.reshape(1, 1),
        W1.T, b1.reshape(1, H),
        W2.T, b2.reshape(1, H),
        W3.T, b3.reshape(1, H),
        Wout.T,
    )
    return y[:, 0]


# speedup vs baseline: 38.6861x; 1.2856x over previous
import functools

import jax
import jax.numpy as jnp
from jax import lax
from jax.experimental import pallas as pl
from jax.experimental.pallas import tpu as pltpu
from jax.experimental.pallas import tpu_sc as plsc

B = 4096
F = 26
V = 1000
D = 16
H = 128

NC = 2
NS = 16
NW = NC * NS
RW = B // NW
GW = RW * F
NCHUNK = GW // 128
RCHUNK = RW // 16


def _sc_body(emb_hbm, fidx_hbm, x_hbm, w_hbm,
             dense_hbm, wsum_hbm,
             idx_v, xv, rows_v, wv, wsum_v, sem):
    c = lax.axis_index("c")
    s = lax.axis_index("s")
    wid = s * NC + c

    pltpu.sync_copy(fidx_hbm.at[wid], idx_v)
    pltpu.sync_copy(x_hbm.at[wid], xv)
    pltpu.sync_copy(w_hbm, wv)

    copies = []
    for j in range(NCHUNK):
        copies.append(
            pltpu.async_copy(emb_hbm.at[idx_v.at[j]],
                             rows_v.at[pl.ds(j * 128, 128)], sem))

    def chunk_body(r, _):
        base = pl.multiple_of(r * (16 * F), 16)
        lane = lax.iota(jnp.int32, 16) * F
        vals = [plsc.load_gather(xv, [lane + (base + f)]) for f in range(F)]
        wsum = jnp.zeros((16,), jnp.float32)
        for f in range(F):
            wf = plsc.load_gather(wv, [vals[f]])
            if f == 0:
                wsum = wf
            else:
                dup = vals[f] == vals[0]
                for f2 in range(1, f):
                    dup = dup | (vals[f] == vals[f2])
                wsum = wsum + jnp.where(dup, 0.0, wf)
        wsum_v[pl.ds(pl.multiple_of(r * 16, 16), 16)] = wsum
        return _

    lax.fori_loop(0, RCHUNK, chunk_body, None)

    for cp in copies:
        cp.wait()
    pltpu.sync_copy(rows_v, dense_hbm.at[pl.ds(pl.multiple_of(wid * GW, 8), GW)])
    pltpu.sync_copy(wsum_v, wsum_hbm.at[pl.ds(pl.multiple_of(wid * RW, 8), RW)])


@jax.jit
def _sc_gather_linear(emb_flat, fidx, xflat, w1k):
    mesh = plsc.VectorSubcoreMesh(core_axis_name="c", subcore_axis_name="s")
    run = pl.kernel(
        _sc_body,
        out_type=(
            jax.ShapeDtypeStruct((B * F, D), jnp.float32),
            jax.ShapeDtypeStruct((B,), jnp.float32),
        ),
        mesh=mesh,
        compiler_params=pltpu.CompilerParams(
            needs_layout_passes=False, use_tc_tiling_on_sc=False),
        scratch_types=[
            pltpu.VMEM((NCHUNK, 128), jnp.int32),
            pltpu.VMEM((GW,), jnp.int32),
            pltpu.VMEM((GW, D), jnp.float32),
            pltpu.VMEM((V,), jnp.float32),
            pltpu.VMEM((RW,), jnp.float32),
            pltpu.SemaphoreType.DMA,
        ],
    )
    return run(emb_flat, fidx, xflat, w1k)


def _tc_body(dense_ref, wsum_ref, fmb_ref, w1_ref, b1_ref, w2_ref, b2_ref,
             w3_ref, b3_ref, wout_ref, out_ref):
    d = dense_ref[:]
    sel = (lax.broadcasted_iota(jnp.int32, (F * D, D), 0) % D
           == lax.broadcasted_iota(jnp.int32, (F * D, D), 1)).astype(jnp.float32)
    s = jnp.dot(d, sel, preferred_element_type=jnp.float32)
    t1 = jnp.dot(s * s, jnp.ones((D, 1), jnp.float32),
                 preferred_element_type=jnp.float32)
    t2 = jnp.dot(d * d, jnp.ones((F * D, 1), jnp.float32),
                 preferred_element_type=jnp.float32)
    pair = 0.5 * (t1 - t2)

    h = jnp.maximum(
        jnp.dot(d, w1_ref[:], preferred_element_type=jnp.float32) + b1_ref[:], 0.0)
    h = jnp.maximum(
        jnp.dot(h, w2_ref[:], preferred_element_type=jnp.float32) + b2_ref[:], 0.0)
    h = jnp.maximum(
        jnp.dot(h, w3_ref[:], preferred_element_type=jnp.float32) + b3_ref[:], 0.0)
    y = jnp.dot(h, wout_ref[:], preferred_element_type=jnp.float32)
    out_ref[:] = y + pair + wsum_ref[:] + fmb_ref[0, 0]


TC_BLK = 1024


@jax.jit
def _tc_fm_mlp(dense2d, wsum2d, fmb2d, W1t, b1r, W2t, b2r, W3t, b3r, Woutt):
    grid = (B // TC_BLK,)
    full = lambda shape: pl.BlockSpec(shape, lambda i: (0, 0))
    return pl.pallas_call(
        _tc_body,
        grid=grid,
        in_specs=[
            pl.BlockSpec((TC_BLK, F * D), lambda i: (i, 0)),
            pl.BlockSpec((TC_BLK, 1), lambda i: (i, 0)),
            full((1, 1)),
            full((F * D, H)),
            full((1, H)),
            full((H, H)),
            full((1, H)),
            full((H, H)),
            full((1, H)),
            full((H, 1)),
        ],
        out_specs=pl.BlockSpec((TC_BLK, 1), lambda i: (i, 0)),
        out_shape=jax.ShapeDtypeStruct((B, 1), jnp.float32),
    )(dense2d, wsum2d, fmb2d, W1t, b1r, W2t, b2r, W3t, b3r, Woutt)


def kernel(x, emb, fm_w, fm_b, W1, b1, W2, b2, W3, b3, Wout):
    offsets = jnp.arange(F, dtype=jnp.int32) * V
    fidx = (x + offsets[None, :]).reshape(NW, NCHUNK, 128)
    xflat = x.reshape(NW, GW)
    emb_flat = emb.reshape(F * V, D)
    w1k = fm_w[0, :V]

    dense, wsum = _sc_gather_linear(emb_flat, fidx, xflat, w1k)

    y = _tc_fm_mlp(
        dense.reshape(B, F * D),
        wsum.reshape(B, 1),
        fm_b.reshape(1, 1),
        W1.T, b1.reshape(1, H),
        W2.T, b2.reshape(1, H),
        W3.T, b3.reshape(1, H),
        Wout.T,
    )
    return y[:, 0]
